# Initial kernel scaffold; baseline (speedup 1.0000x reference)
import functools

import jax
import jax.numpy as jnp
from jax.experimental import pallas as pl
from jax.experimental.pallas import tpu as pltpu
from jax.experimental.pallas import tpu_sc as plsc

B, S, V, H = 64, 512, 30522, 768
N_TOK = B * S
TWOD = 1024
LN_EPS = 1e-12

GW = 32
TB = 512


def _sc_word_gather(weight, ids2d):
    @functools.partial(
        pl.kernel,
        out_type=jax.ShapeDtypeStruct((N_TOK, H), weight.dtype),
        mesh=plsc.VectorSubcoreMesh(
            core_axis_name="c", subcore_axis_name="s", num_cores=2,
            num_subcores=16),
    )
    def gather_kernel(w_hbm, i_hbm, o_hbm):
        def body(i_vmem, o_vmem):
            pltpu.sync_copy(w_hbm.at[i_vmem.at[0]], o_vmem)

        pltpu.emit_pipeline(
            body,
            grid=(N_TOK // GW,),
            in_specs=[pl.BlockSpec((1, GW), lambda i: (0, i))],
            out_specs=[pl.BlockSpec((GW, H), lambda i: (i, 0))],
            core_axis_name=("c", "s"),
            dimension_semantics=(pltpu.PARALLEL,),
        )(i_hbm, o_hbm)

    return gather_kernel(weight, ids2d)


def _tc_body(bbox_ref, g_ref, x_ref, y_ref, h_ref, w_ref, p_ref, t_ref,
             gam_ref, bet_ref, o_ref):
    bb = bbox_ref[...]
    left = bb[:, 0:1]
    upper = bb[:, 1:2]
    right = bb[:, 2:3]
    lower = bb[:, 3:4]
    hh = lower - upper
    ww = right - left
    iota = jax.lax.broadcasted_iota(jnp.int32, (TB, TWOD), 1)

    def onehot(idx):
        return (iota == idx).astype(jnp.bfloat16)

    cx = onehot(left) + onehot(right)
    cy = onehot(upper) + onehot(lower)
    ch = onehot(hh)
    cw = onehot(ww)

    acc = g_ref[...] + p_ref[...] + t_ref[0:1, :]
    acc = acc + jnp.dot(cx, x_ref[...], preferred_element_type=jnp.float32)
    acc = acc + jnp.dot(cy, y_ref[...], preferred_element_type=jnp.float32)
    acc = acc + jnp.dot(ch, h_ref[...], preferred_element_type=jnp.float32)
    acc = acc + jnp.dot(cw, w_ref[...], preferred_element_type=jnp.float32)

    mean = jnp.mean(acc, axis=1, keepdims=True)
    cen = acc - mean
    var = jnp.mean(cen * cen, axis=1, keepdims=True)
    o_ref[...] = (cen * jax.lax.rsqrt(var + LN_EPS)) * gam_ref[...] + bet_ref[...]


def _tc_combine(bbox2, gathered, xb, yb, hb, wb, pe, tte, gam2, bet2):
    return pl.pallas_call(
        _tc_body,
        grid=(N_TOK // TB,),
        in_specs=[
            pl.BlockSpec((TB, 4), lambda i: (i, 0)),
            pl.BlockSpec((TB, H), lambda i: (i, 0)),
            pl.BlockSpec((TWOD, H), lambda i: (0, 0)),
            pl.BlockSpec((TWOD, H), lambda i: (0, 0)),
            pl.BlockSpec((TWOD, H), lambda i: (0, 0)),
            pl.BlockSpec((TWOD, H), lambda i: (0, 0)),
            pl.BlockSpec((S, H), lambda i: (0, 0)),
            pl.BlockSpec((2, H), lambda i: (0, 0)),
            pl.BlockSpec((1, H), lambda i: (0, 0)),
            pl.BlockSpec((1, H), lambda i: (0, 0)),
        ],
        out_specs=pl.BlockSpec((TB, H), lambda i: (i, 0)),
        out_shape=jax.ShapeDtypeStruct((N_TOK, H), jnp.float32),
        compiler_params=pltpu.CompilerParams(
            dimension_semantics=("arbitrary",)),
    )(bbox2, gathered, xb, yb, hb, wb, pe, tte, gam2, bet2)


def kernel(input_ids, bbox, weight, token_type_embeddings,
           position_embeddings, x_position_embeddings, y_position_embeddings,
           h_position_embeddings, w_position_embeddings, ln_gamma, ln_beta):
    ids2d = input_ids.reshape(1, N_TOK)
    gathered = _sc_word_gather(weight, ids2d)

    bbox2 = bbox.reshape(N_TOK, 4)
    xb = x_position_embeddings.astype(jnp.bfloat16)
    yb = y_position_embeddings.astype(jnp.bfloat16)
    hb = h_position_embeddings.astype(jnp.bfloat16)
    wb = w_position_embeddings.astype(jnp.bfloat16)
    gam2 = ln_gamma.reshape(1, H)
    bet2 = ln_beta.reshape(1, H)

    out = _tc_combine(bbox2, gathered, xb, yb, hb, wb,
                      position_embeddings, token_type_embeddings, gam2, bet2)
    return out.reshape(B, S, H)


# speedup vs baseline: 3.8651x; 3.8651x over previous
import functools

import jax
import jax.numpy as jnp
from jax.experimental import pallas as pl
from jax.experimental.pallas import tpu as pltpu
from jax.experimental.pallas import tpu_sc as plsc

B, S, V, H = 64, 512, 30522, 768
N_TOK = B * S
TWOD = 1024
LN_EPS = 1e-12

NC, NS = 2, 16
NW = NC * NS
B_PER_W = N_TOK // NW
CH = 64
NCH = B_PER_W // CH
TB = 512


def _sc_word_gather(weight, ids1d):
    @functools.partial(
        pl.kernel,
        out_type=jax.ShapeDtypeStruct((N_TOK, H), weight.dtype),
        mesh=plsc.VectorSubcoreMesh(
            core_axis_name="c", subcore_axis_name="s", num_cores=NC,
            num_subcores=NS),
        scratch_types=[
            pltpu.VMEM((B_PER_W,), jnp.int32),
            pltpu.VMEM((CH, H), jnp.float32),
            pltpu.SemaphoreType.DMA,
        ],
    )
    def gather_kernel(w_hbm, i_hbm, o_hbm, idx_v, rows_v, sem):
        wid = jax.lax.axis_index("s") * NC + jax.lax.axis_index("c")
        base = wid * B_PER_W
        pltpu.sync_copy(i_hbm.at[pl.ds(base, B_PER_W)], idx_v)

        @pl.loop(0, NCH)
        def _(i):
            pltpu.async_copy(
                w_hbm.at[idx_v.at[pl.ds(i * CH, CH)]], rows_v, sem).wait()
            pltpu.sync_copy(rows_v, o_hbm.at[pl.ds(base + i * CH, CH)])

    return gather_kernel(weight, ids1d)


def _tc_body(bbox_ref, g_ref, x_ref, y_ref, h_ref, w_ref, p_ref, t_ref,
             gam_ref, bet_ref, o_ref):
    bb = bbox_ref[...]
    left = bb[:, 0:1]
    upper = bb[:, 1:2]
    right = bb[:, 2:3]
    lower = bb[:, 3:4]
    hh = lower - upper
    ww = right - left
    iota = jax.lax.broadcasted_iota(jnp.int32, (TB, TWOD), 1)

    def onehot(idx):
        return (iota == idx).astype(jnp.bfloat16)

    cx = onehot(left) + onehot(right)
    cy = onehot(upper) + onehot(lower)
    ch = onehot(hh)
    cw = onehot(ww)

    acc = g_ref[...] + p_ref[...] + t_ref[0:1, :]
    acc = acc + jnp.dot(cx, x_ref[...], preferred_element_type=jnp.float32)
    acc = acc + jnp.dot(cy, y_ref[...], preferred_element_type=jnp.float32)
    acc = acc + jnp.dot(ch, h_ref[...], preferred_element_type=jnp.float32)
    acc = acc + jnp.dot(cw, w_ref[...], preferred_element_type=jnp.float32)

    mean = jnp.mean(acc, axis=1, keepdims=True)
    cen = acc - mean
    var = jnp.mean(cen * cen, axis=1, keepdims=True)
    o_ref[...] = (cen * jax.lax.rsqrt(var + LN_EPS)) * gam_ref[...] + bet_ref[...]


def _tc_combine(bbox2, gathered, xb, yb, hb, wb, pe, tte, gam2, bet2):
    return pl.pallas_call(
        _tc_body,
        grid=(N_TOK // TB,),
        in_specs=[
            pl.BlockSpec((TB, 4), lambda i: (i, 0)),
            pl.BlockSpec((TB, H), lambda i: (i, 0)),
            pl.BlockSpec((TWOD, H), lambda i: (0, 0)),
            pl.BlockSpec((TWOD, H), lambda i: (0, 0)),
            pl.BlockSpec((TWOD, H), lambda i: (0, 0)),
            pl.BlockSpec((TWOD, H), lambda i: (0, 0)),
            pl.BlockSpec((S, H), lambda i: (0, 0)),
            pl.BlockSpec((2, H), lambda i: (0, 0)),
            pl.BlockSpec((1, H), lambda i: (0, 0)),
            pl.BlockSpec((1, H), lambda i: (0, 0)),
        ],
        out_specs=pl.BlockSpec((TB, H), lambda i: (i, 0)),
        out_shape=jax.ShapeDtypeStruct((N_TOK, H), jnp.float32),
        compiler_params=pltpu.CompilerParams(
            dimension_semantics=("arbitrary",)),
    )(bbox2, gathered, xb, yb, hb, wb, pe, tte, gam2, bet2)


def kernel(input_ids, bbox, weight, token_type_embeddings,
           position_embeddings, x_position_embeddings, y_position_embeddings,
           h_position_embeddings, w_position_embeddings, ln_gamma, ln_beta):
    ids1d = input_ids.reshape(N_TOK)
    gathered = _sc_word_gather(weight, ids1d)

    bbox2 = bbox.reshape(N_TOK, 4)
    xb = x_position_embeddings.astype(jnp.bfloat16)
    yb = y_position_embeddings.astype(jnp.bfloat16)
    hb = h_position_embeddings.astype(jnp.bfloat16)
    wb = w_position_embeddings.astype(jnp.bfloat16)
    gam2 = ln_gamma.reshape(1, H)
    bet2 = ln_beta.reshape(1, H)

    out = _tc_combine(bbox2, gathered, xb, yb, hb, wb,
                      position_embeddings, token_type_embeddings, gam2, bet2)
    return out.reshape(B, S, H)


# speedup vs baseline: 3.8724x; 1.0019x over previous
import functools

import jax
import jax.numpy as jnp
from jax.experimental import pallas as pl
from jax.experimental.pallas import tpu as pltpu
from jax.experimental.pallas import tpu_sc as plsc

B, S, V, H = 64, 512, 30522, 768
N_TOK = B * S
TWOD = 1024
LN_EPS = 1e-12

NC, NS = 2, 16
NW = NC * NS
B_PER_W = N_TOK // NW
CH = 64
NCH = B_PER_W // CH
TB = 512


def _sc_word_gather(weight, ids1d):
    @functools.partial(
        pl.kernel,
        out_type=jax.ShapeDtypeStruct((N_TOK, H), weight.dtype),
        mesh=plsc.VectorSubcoreMesh(
            core_axis_name="c", subcore_axis_name="s", num_cores=NC,
            num_subcores=NS),
        scratch_types=[
            pltpu.VMEM((B_PER_W,), jnp.int32),
            pltpu.VMEM((CH, H), jnp.float32),
            pltpu.SemaphoreType.DMA,
        ],
    )
    def gather_kernel(w_hbm, i_hbm, o_hbm, idx_v, rows_v, sem):
        wid = jax.lax.axis_index("s") * NC + jax.lax.axis_index("c")
        base = wid * B_PER_W
        pltpu.sync_copy(i_hbm.at[pl.ds(base, B_PER_W)], idx_v)

        @pl.loop(0, NCH)
        def _(i):
            pltpu.async_copy(
                w_hbm.at[idx_v.at[pl.ds(i * CH, CH)]], rows_v, sem).wait()
            pltpu.sync_copy(rows_v, o_hbm.at[pl.ds(base + i * CH, CH)])

    return gather_kernel(weight, ids1d)


def _tc_body(bbox_ref, g_ref, x_ref, y_ref, h_ref, w_ref, p_ref, t_ref,
             gam_ref, bet_ref, o_ref):
    bb = bbox_ref[...]
    left = bb[:, 0:1]
    upper = bb[:, 1:2]
    right = bb[:, 2:3]
    lower = bb[:, 3:4]
    hh = lower - upper
    ww = right - left
    iota = jax.lax.broadcasted_iota(jnp.int32, (TB, TWOD), 1)

    def onehot(idx):
        return (iota == idx).astype(jnp.bfloat16)

    cx = onehot(left) + onehot(right)
    cy = onehot(upper) + onehot(lower)
    ch = onehot(hh)
    cw = onehot(ww)

    acc = g_ref[...] + p_ref[...] + t_ref[0:1, :]
    acc = acc + jnp.dot(cx, x_ref[...], preferred_element_type=jnp.float32)
    acc = acc + jnp.dot(cy, y_ref[...], preferred_element_type=jnp.float32)
    acc = acc + jnp.dot(ch, h_ref[...], preferred_element_type=jnp.float32)
    acc = acc + jnp.dot(cw, w_ref[...], preferred_element_type=jnp.float32)

    mean = jnp.mean(acc, axis=1, keepdims=True)
    cen = acc - mean
    var = jnp.mean(cen * cen, axis=1, keepdims=True)
    o_ref[...] = (cen * jax.lax.rsqrt(var + LN_EPS)) * gam_ref[...] + bet_ref[...]


def _tc_combine(bbox2, gathered, xb, yb, hb, wb, pe, tte, gam2, bet2):
    return pl.pallas_call(
        _tc_body,
        grid=(N_TOK // TB,),
        in_specs=[
            pl.BlockSpec((TB, 4), lambda i: (i, 0)),
            pl.BlockSpec((TB, H), lambda i: (i, 0)),
            pl.BlockSpec((TWOD, H), lambda i: (0, 0)),
            pl.BlockSpec((TWOD, H), lambda i: (0, 0)),
            pl.BlockSpec((TWOD, H), lambda i: (0, 0)),
            pl.BlockSpec((TWOD, H), lambda i: (0, 0)),
            pl.BlockSpec((S, H), lambda i: (0, 0)),
            pl.BlockSpec((2, H), lambda i: (0, 0)),
            pl.BlockSpec((1, H), lambda i: (0, 0)),
            pl.BlockSpec((1, H), lambda i: (0, 0)),
        ],
        out_specs=pl.BlockSpec((TB, H), lambda i: (i, 0)),
        out_shape=jax.ShapeDtypeStruct((N_TOK, H), jnp.float32),
        compiler_params=pltpu.CompilerParams(
            dimension_semantics=("parallel",)),
    )(bbox2, gathered, xb, yb, hb, wb, pe, tte, gam2, bet2)


def kernel(input_ids, bbox, weight, token_type_embeddings,
           position_embeddings, x_position_embeddings, y_position_embeddings,
           h_position_embeddings, w_position_embeddings, ln_gamma, ln_beta):
    ids1d = input_ids.reshape(N_TOK)
    gathered = _sc_word_gather(weight, ids1d)

    bbox2 = bbox.reshape(N_TOK, 4)
    xb = x_position_embeddings.astype(jnp.bfloat16)
    yb = y_position_embeddings.astype(jnp.bfloat16)
    hb = h_position_embeddings.astype(jnp.bfloat16)
    wb = w_position_embeddings.astype(jnp.bfloat16)
    gam2 = ln_gamma.reshape(1, H)
    bet2 = ln_beta.reshape(1, H)

    out = _tc_combine(bbox2, gathered, xb, yb, hb, wb,
                      position_embeddings, token_type_embeddings, gam2, bet2)
    return out.reshape(B, S, H)
